# baseline (speedup 1.0000x reference)
import functools

import jax
import jax.numpy as jnp
from jax import lax
from jax.experimental import pallas as pl
from jax.experimental.pallas import tpu as pltpu
from jax.experimental.pallas import tpu_sc as plsc

HIDDEN = 2048
FFN = 8192
E = 8
S = 2048
CAP = 512
NSLOT = E * CAP
DUMP = NSLOT
DISP_ROWS = NSLOT + 8
NEG = -1e30

NW = 32
TW = S // NW


def _routing_body(xf_ref, wg_ref, small_ref, g1_ref, g2_ref,
                  s1c_ref, s2c_ref, s1d_ref, s2d_ref):
    xf = xf_ref[...]
    logits = jnp.dot(xf, wg_ref[...], preferred_element_type=jnp.float32)

    m = jnp.max(logits, axis=1, keepdims=True)
    ex = jnp.exp(logits - m)
    gates = ex / jnp.sum(ex, axis=1, keepdims=True)

    uppertri = (jax.lax.broadcasted_iota(jnp.int32, (E, E), 0)
                < jax.lax.broadcasted_iota(jnp.int32, (E, E), 1)).astype(jnp.float32)
    eq1 = (logits == m).astype(jnp.float32)
    before1 = jnp.dot(eq1, uppertri, preferred_element_type=jnp.float32)
    mask1 = eq1 * (before1 == 0.0).astype(jnp.float32)

    logits2 = jnp.where(mask1 > 0, NEG, logits)
    m2 = jnp.max(logits2, axis=1, keepdims=True)
    eq2 = (logits2 == m2).astype(jnp.float32)
    before2 = jnp.dot(eq2, uppertri, preferred_element_type=jnp.float32)
    mask2 = eq2 * (before2 == 0.0).astype(jnp.float32)

    lower = (jax.lax.broadcasted_iota(jnp.int32, (S, S), 0)
             >= jax.lax.broadcasted_iota(jnp.int32, (S, S), 1)).astype(jnp.float32)
    locations1 = jnp.dot(lower, mask1, preferred_element_type=jnp.float32) - 1.0
    count1 = jnp.sum(mask1, axis=0, keepdims=True)
    locations2 = (jnp.dot(lower, mask2, preferred_element_type=jnp.float32)
                  - 1.0 + count1)

    me = jnp.mean(gates, axis=0)
    ce = jnp.mean(mask1, axis=0)
    l_aux = jnp.mean(me * ce) * (E * E)

    mask1c = mask1 * (locations1 < CAP).astype(jnp.float32)
    mask2c = mask2 * (locations2 < CAP).astype(jnp.float32)
    loc1 = jnp.sum(locations1 * mask1c, axis=1, keepdims=True)
    loc2 = jnp.sum(locations2 * mask2c, axis=1, keepdims=True)
    g1 = jnp.sum(gates * mask1c, axis=1, keepdims=True)
    g2 = jnp.sum(gates * mask2c, axis=1, keepdims=True)
    denom = jnp.maximum(g1 + g2, jnp.finfo(jnp.float32).eps)
    g1_ref[...] = jnp.broadcast_to(g1 / denom, (S, 16))
    g2_ref[...] = jnp.broadcast_to(g2 / denom, (S, 16))

    lane = jax.lax.broadcasted_iota(jnp.int32, (S, E), 1).astype(jnp.float32)
    idx1 = jnp.sum(lane * mask1, axis=1, keepdims=True)
    idx2 = jnp.sum(lane * mask2, axis=1, keepdims=True)
    keep1 = jnp.sum(mask1c, axis=1, keepdims=True)
    keep2 = jnp.sum(mask2c, axis=1, keepdims=True)
    s1c = idx1 * CAP + loc1
    s2c = idx2 * CAP + loc2
    s1c_ref[...] = s1c
    s2c_ref[...] = s2c
    s1d_ref[...] = jnp.where(keep1 > 0, s1c, float(DUMP))
    s2d_ref[...] = jnp.where(keep2 > 0, s2c, float(DUMP))

    cnt_pad = jnp.concatenate(
        [count1, jnp.zeros((1, 128 - E), jnp.float32)], axis=1)
    row = jax.lax.broadcasted_iota(jnp.int32, (8, 128), 0)
    col = jax.lax.broadcasted_iota(jnp.int32, (8, 128), 1)
    small = jnp.where(row == 0, jnp.broadcast_to(cnt_pad, (8, 128)), 0.0)
    small = jnp.where((row == 1) & (col == 0), l_aux, small)
    small_ref[...] = small


def _dispatch_body(xf_hbm, s1d_hbm, s2d_hbm, disp_hbm,
                   xrows_v, i1_v, i2_v, gsem0, gsem1, ssem0, ssem1):
    wid = lax.axis_index("s") * 2 + lax.axis_index("c")
    nsub = 4
    sub_n = TW // nsub
    base = wid * TW
    gsem = (gsem0, gsem1)
    ssem = (ssem0, ssem1)
    pltpu.sync_copy(s1d_hbm.at[pl.ds(wid * nsub, nsub)], i1_v)
    pltpu.sync_copy(s2d_hbm.at[pl.ds(wid * nsub, nsub)], i2_v)

    def load(sub, buf):
        return pltpu.async_copy(
            xf_hbm.at[pl.ds(base + sub * sub_n, sub_n)],
            xrows_v.at[buf], gsem[buf])

    load_h = [load(0, 0), None]
    store_h = [None, None]
    for sub in range(nsub):
        buf = sub & 1
        if sub + 1 < nsub:
            ob = (sub + 1) & 1
            if store_h[ob] is not None:
                for st in store_h[ob]:
                    st.wait()
                store_h[ob] = None
            load_h[ob] = load(sub + 1, ob)
        load_h[buf].wait()
        store_h[buf] = (
            pltpu.async_copy(xrows_v.at[buf],
                             disp_hbm.at[i1_v.at[sub]], ssem[buf]),
            pltpu.async_copy(xrows_v.at[buf],
                             disp_hbm.at[i2_v.at[sub]], ssem[buf]),
        )
    for pair in store_h:
        if pair is not None:
            for st in pair:
                st.wait()


def _ffn_body(nf, disp_ref, w1_ref, w3_ref, w2_ref, eos_ref, acc):
    f = pl.program_id(1)

    d = disp_ref[...].astype(jnp.bfloat16)
    a = jnp.dot(d, w1_ref[0].astype(jnp.bfloat16),
                preferred_element_type=jnp.float32)
    b = jnp.dot(d, w3_ref[0].astype(jnp.bfloat16),
                preferred_element_type=jnp.float32)
    h = (a * jax.nn.sigmoid(a) * b).astype(jnp.bfloat16)
    contrib = jnp.dot(h, w2_ref[0].astype(jnp.bfloat16),
                      preferred_element_type=jnp.float32)

    @pl.when(f == 0)
    def _():
        acc[...] = jnp.zeros_like(acc)

    acc[...] += contrib

    @pl.when(f == nf - 1)
    def _():
        eos_ref[...] = acc[...]


def _combine_body(eos_hbm, s1c_hbm, s2c_hbm, g1_hbm, g2_hbm, out_hbm,
                  r1_v, r2_v, i1_v, i2_v, g1_v, g2_v,
                  gsem0, gsem1, ssem0, ssem1):
    wid = lax.axis_index("s") * 2 + lax.axis_index("c")
    nchunk = TW // 8
    gsem = (gsem0, gsem1)
    ssem = (ssem0, ssem1)
    base = wid * TW
    pltpu.sync_copy(s1c_hbm.at[pl.ds(wid * nchunk, nchunk)], i1_v)
    pltpu.sync_copy(s2c_hbm.at[pl.ds(wid * nchunk, nchunk)], i2_v)
    pltpu.sync_copy(g1_hbm.at[pl.ds(base, TW)], g1_v)
    pltpu.sync_copy(g2_hbm.at[pl.ds(base, TW)], g2_v)

    def gather(chunk, buf):
        return (pltpu.async_copy(eos_hbm.at[i1_v.at[chunk]],
                                 r1_v.at[buf], gsem[buf]),
                pltpu.async_copy(eos_hbm.at[i2_v.at[chunk]],
                                 r2_v.at[buf], gsem[buf]))

    gath_h = [gather(0, 0), None]
    store_h = [None, None]
    for chunk in range(nchunk):
        buf = chunk & 1
        if chunk + 1 < nchunk:
            ob = (chunk + 1) & 1
            if store_h[ob] is not None:
                store_h[ob].wait()
                store_h[ob] = None
            gath_h[ob] = gather(chunk + 1, ob)
        gath_h[buf][0].wait()
        gath_h[buf][1].wait()

        def body_i(i, carry):
            bg1 = g1_v[chunk * 8 + i, :]
            bg2 = g2_v[chunk * 8 + i, :]

            def body_j(j, c2):
                for u in range(32):
                    sl = pl.ds(j * 512 + u * 16, 16)
                    r1_v[buf, i, sl] = (bg1 * r1_v[buf, i, sl]
                                        + bg2 * r2_v[buf, i, sl])
                return c2

            return lax.fori_loop(0, HIDDEN // 512, body_j, carry)

        lax.fori_loop(0, 8, body_i, 0)
        store_h[buf] = pltpu.async_copy(
            r1_v.at[buf], out_hbm.at[pl.ds(base + chunk * 8, 8)], ssem[buf])
    for st in store_h:
        if st is not None:
            st.wait()


def kernel(hidden_states, wg, w1, w3, w2):
    B, SS, H = hidden_states.shape
    xf = hidden_states.reshape(S, H)

    small, g1, g2, s1c, s2c, s1d, s2d = pl.pallas_call(
        _routing_body,
        out_shape=(
            jax.ShapeDtypeStruct((8, 128), jnp.float32),
            jax.ShapeDtypeStruct((S, 16), jnp.float32),
            jax.ShapeDtypeStruct((S, 16), jnp.float32),
            jax.ShapeDtypeStruct((S, 1), jnp.float32),
            jax.ShapeDtypeStruct((S, 1), jnp.float32),
            jax.ShapeDtypeStruct((S, 1), jnp.float32),
            jax.ShapeDtypeStruct((S, 1), jnp.float32),
        ),
    )(xf, wg)

    s1ci = s1c.reshape(S // 8, 8).astype(jnp.int32)
    s2ci = s2c.reshape(S // 8, 8).astype(jnp.int32)
    s1di = s1d.reshape(S // 16, 16).astype(jnp.int32)
    s2di = s2d.reshape(S // 16, 16).astype(jnp.int32)

    mesh = plsc.VectorSubcoreMesh(core_axis_name="c", subcore_axis_name="s")
    disp = pl.kernel(
        _dispatch_body,
        out_type=jax.ShapeDtypeStruct((DISP_ROWS, H), jnp.float32),
        mesh=mesh,
        scratch_types=[
            pltpu.VMEM((2, TW // 4, H), jnp.float32),
            pltpu.VMEM((4, 16), jnp.int32),
            pltpu.VMEM((4, 16), jnp.int32),
            pltpu.SemaphoreType.DMA,
            pltpu.SemaphoreType.DMA,
            pltpu.SemaphoreType.DMA,
            pltpu.SemaphoreType.DMA,
        ],
    )(xf, s1di, s2di)

    NF = 16
    FBLK = FFN // NF
    eos = pl.pallas_call(
        functools.partial(_ffn_body, NF),
        grid=(E, NF),
        in_specs=[
            pl.BlockSpec((CAP, H), lambda e, f: (e, 0)),
            pl.BlockSpec((1, H, FBLK), lambda e, f: (e, 0, f)),
            pl.BlockSpec((1, H, FBLK), lambda e, f: (e, 0, f)),
            pl.BlockSpec((1, FBLK, H), lambda e, f: (e, f, 0)),
        ],
        out_specs=pl.BlockSpec((CAP, H), lambda e, f: (e, 0)),
        out_shape=jax.ShapeDtypeStruct((NSLOT, H), jnp.float32),
        scratch_shapes=[
            pltpu.VMEM((CAP, H), jnp.float32),
        ],
    )(disp, w1, w3, w2)

    out = pl.kernel(
        _combine_body,
        out_type=jax.ShapeDtypeStruct((S, H), jnp.float32),
        mesh=mesh,
        scratch_types=[
            pltpu.VMEM((2, 8, H), jnp.float32),
            pltpu.VMEM((2, 8, H), jnp.float32),
            pltpu.VMEM((8, 8), jnp.int32),
            pltpu.VMEM((8, 8), jnp.int32),
            pltpu.VMEM((TW, 16), jnp.float32),
            pltpu.VMEM((TW, 16), jnp.float32),
            pltpu.SemaphoreType.DMA,
            pltpu.SemaphoreType.DMA,
            pltpu.SemaphoreType.DMA,
            pltpu.SemaphoreType.DMA,
        ],
    )(eos, s1ci, s2ci, g1, g2)

    l_aux = small[1, 0]
    exp_counts = small[0, :E].astype(jnp.int32)
    return out.reshape(B, SS, H), l_aux, exp_counts


# speedup vs baseline: 1.0111x; 1.0111x over previous
import functools

import jax
import jax.numpy as jnp
from jax import lax
from jax.experimental import pallas as pl
from jax.experimental.pallas import tpu as pltpu
from jax.experimental.pallas import tpu_sc as plsc

HIDDEN = 2048
FFN = 8192
E = 8
S = 2048
CAP = 512
NSLOT = E * CAP
DUMP = NSLOT
DISP_ROWS = NSLOT + 8
NEG = -1e30

NW = 32
TW = S // NW


def _routing_body(xf_ref, wg_ref, small_ref, g1_ref, g2_ref,
                  s1c_ref, s2c_ref, s1d_ref, s2d_ref):
    xf = xf_ref[...]
    logits = jnp.dot(xf, wg_ref[...], preferred_element_type=jnp.float32)

    m = jnp.max(logits, axis=1, keepdims=True)
    ex = jnp.exp(logits - m)
    gates = ex / jnp.sum(ex, axis=1, keepdims=True)

    uppertri = (jax.lax.broadcasted_iota(jnp.int32, (E, E), 0)
                < jax.lax.broadcasted_iota(jnp.int32, (E, E), 1)).astype(jnp.float32)
    eq1 = (logits == m).astype(jnp.float32)
    before1 = jnp.dot(eq1, uppertri, preferred_element_type=jnp.float32)
    mask1 = eq1 * (before1 == 0.0).astype(jnp.float32)

    logits2 = jnp.where(mask1 > 0, NEG, logits)
    m2 = jnp.max(logits2, axis=1, keepdims=True)
    eq2 = (logits2 == m2).astype(jnp.float32)
    before2 = jnp.dot(eq2, uppertri, preferred_element_type=jnp.float32)
    mask2 = eq2 * (before2 == 0.0).astype(jnp.float32)

    lower = (jax.lax.broadcasted_iota(jnp.int32, (S, S), 0)
             >= jax.lax.broadcasted_iota(jnp.int32, (S, S), 1)).astype(jnp.float32)
    locations1 = jnp.dot(lower, mask1, preferred_element_type=jnp.float32) - 1.0
    count1 = jnp.sum(mask1, axis=0, keepdims=True)
    locations2 = (jnp.dot(lower, mask2, preferred_element_type=jnp.float32)
                  - 1.0 + count1)

    me = jnp.mean(gates, axis=0)
    ce = jnp.mean(mask1, axis=0)
    l_aux = jnp.mean(me * ce) * (E * E)

    mask1c = mask1 * (locations1 < CAP).astype(jnp.float32)
    mask2c = mask2 * (locations2 < CAP).astype(jnp.float32)
    loc1 = jnp.sum(locations1 * mask1c, axis=1, keepdims=True)
    loc2 = jnp.sum(locations2 * mask2c, axis=1, keepdims=True)
    g1 = jnp.sum(gates * mask1c, axis=1, keepdims=True)
    g2 = jnp.sum(gates * mask2c, axis=1, keepdims=True)
    denom = jnp.maximum(g1 + g2, jnp.finfo(jnp.float32).eps)
    g1_ref[...] = jnp.broadcast_to(g1 / denom, (S, 16))
    g2_ref[...] = jnp.broadcast_to(g2 / denom, (S, 16))

    lane = jax.lax.broadcasted_iota(jnp.int32, (S, E), 1).astype(jnp.float32)
    idx1 = jnp.sum(lane * mask1, axis=1, keepdims=True)
    idx2 = jnp.sum(lane * mask2, axis=1, keepdims=True)
    keep1 = jnp.sum(mask1c, axis=1, keepdims=True)
    keep2 = jnp.sum(mask2c, axis=1, keepdims=True)
    s1c = idx1 * CAP + loc1
    s2c = idx2 * CAP + loc2
    s1c_ref[...] = s1c
    s2c_ref[...] = s2c
    s1d_ref[...] = jnp.where(keep1 > 0, s1c, float(DUMP))
    s2d_ref[...] = jnp.where(keep2 > 0, s2c, float(DUMP))

    cnt_pad = jnp.concatenate(
        [count1, jnp.zeros((1, 128 - E), jnp.float32)], axis=1)
    row = jax.lax.broadcasted_iota(jnp.int32, (8, 128), 0)
    col = jax.lax.broadcasted_iota(jnp.int32, (8, 128), 1)
    small = jnp.where(row == 0, jnp.broadcast_to(cnt_pad, (8, 128)), 0.0)
    small = jnp.where((row == 1) & (col == 0), l_aux, small)
    small_ref[...] = small


def _dispatch_body(xf_hbm, s1d_hbm, s2d_hbm, disp_hbm,
                   xrows_v, i1_v, i2_v, gsem0, gsem1, ssem0, ssem1):
    wid = lax.axis_index("s") * 2 + lax.axis_index("c")
    nsub = 4
    sub_n = TW // nsub
    base = wid * TW
    gsem = (gsem0, gsem1)
    ssem = (ssem0, ssem1)
    pltpu.sync_copy(s1d_hbm.at[pl.ds(wid * nsub, nsub)], i1_v)
    pltpu.sync_copy(s2d_hbm.at[pl.ds(wid * nsub, nsub)], i2_v)

    def load(sub, buf):
        return pltpu.async_copy(
            xf_hbm.at[pl.ds(base + sub * sub_n, sub_n)],
            xrows_v.at[buf], gsem[buf])

    load_h = [load(0, 0), None]
    store_h = [None, None]
    for sub in range(nsub):
        buf = sub & 1
        if sub + 1 < nsub:
            ob = (sub + 1) & 1
            if store_h[ob] is not None:
                for st in store_h[ob]:
                    st.wait()
                store_h[ob] = None
            load_h[ob] = load(sub + 1, ob)
        load_h[buf].wait()
        store_h[buf] = (
            pltpu.async_copy(xrows_v.at[buf],
                             disp_hbm.at[i1_v.at[sub]], ssem[buf]),
            pltpu.async_copy(xrows_v.at[buf],
                             disp_hbm.at[i2_v.at[sub]], ssem[buf]),
        )
    for pair in store_h:
        if pair is not None:
            for st in pair:
                st.wait()


def _ffn_body(nf, disp_ref, w1_ref, w3_ref, w2_ref, eos_ref, acc):
    f = pl.program_id(1)

    d = disp_ref[...].astype(jnp.bfloat16)
    a = jnp.dot(d, w1_ref[0].astype(jnp.bfloat16),
                preferred_element_type=jnp.float32)
    b = jnp.dot(d, w3_ref[0].astype(jnp.bfloat16),
                preferred_element_type=jnp.float32)
    h = (a * jax.nn.sigmoid(a) * b).astype(jnp.bfloat16)
    contrib = jnp.dot(h, w2_ref[0].astype(jnp.bfloat16),
                      preferred_element_type=jnp.float32)

    @pl.when(f == 0)
    def _():
        acc[...] = jnp.zeros_like(acc)

    acc[...] += contrib

    @pl.when(f == nf - 1)
    def _():
        eos_ref[...] = acc[...]


def _combine_body(eos_hbm, s1c_hbm, s2c_hbm, g1_hbm, g2_hbm, out_hbm,
                  r1_v, r2_v, i1_v, i2_v, g1_v, g2_v, sem):
    wid = lax.axis_index("s") * 2 + lax.axis_index("c")
    nchunk = TW // 16
    for chunk in range(nchunk):
        tb = wid * TW + chunk * 16
        pltpu.sync_copy(s1c_hbm.at[pl.ds(tb, 16)], i1_v)
        pltpu.sync_copy(s2c_hbm.at[pl.ds(tb, 16)], i2_v)
        pltpu.sync_copy(g1_hbm.at[pl.ds(tb, 16)], g1_v)
        pltpu.sync_copy(g2_hbm.at[pl.ds(tb, 16)], g2_v)
        gt1 = pltpu.async_copy(eos_hbm.at[i1_v], r1_v, sem)
        gt2 = pltpu.async_copy(eos_hbm.at[i2_v], r2_v, sem)
        gt1.wait()
        gt2.wait()

        def body_i(i, carry):
            bg1 = g1_v[i, :]
            bg2 = g2_v[i, :]
            for u in range(HIDDEN // 16):
                sl = pl.ds(u * 16, 16)
                r1_v[i, sl] = bg1 * r1_v[i, sl] + bg2 * r2_v[i, sl]
            return carry

        lax.fori_loop(0, 16, body_i, 0)
        pltpu.sync_copy(r1_v, out_hbm.at[pl.ds(tb, 16)])


def kernel(hidden_states, wg, w1, w3, w2):
    B, SS, H = hidden_states.shape
    xf = hidden_states.reshape(S, H)

    small, g1, g2, s1c, s2c, s1d, s2d = pl.pallas_call(
        _routing_body,
        out_shape=(
            jax.ShapeDtypeStruct((8, 128), jnp.float32),
            jax.ShapeDtypeStruct((S, 16), jnp.float32),
            jax.ShapeDtypeStruct((S, 16), jnp.float32),
            jax.ShapeDtypeStruct((S, 1), jnp.float32),
            jax.ShapeDtypeStruct((S, 1), jnp.float32),
            jax.ShapeDtypeStruct((S, 1), jnp.float32),
            jax.ShapeDtypeStruct((S, 1), jnp.float32),
        ),
    )(xf, wg)

    s1ci = s1c.reshape(S).astype(jnp.int32)
    s2ci = s2c.reshape(S).astype(jnp.int32)
    s1di = s1d.reshape(S // 16, 16).astype(jnp.int32)
    s2di = s2d.reshape(S // 16, 16).astype(jnp.int32)

    mesh = plsc.VectorSubcoreMesh(core_axis_name="c", subcore_axis_name="s")
    disp = pl.kernel(
        _dispatch_body,
        out_type=jax.ShapeDtypeStruct((DISP_ROWS, H), jnp.float32),
        mesh=mesh,
        scratch_types=[
            pltpu.VMEM((2, TW // 4, H), jnp.float32),
            pltpu.VMEM((4, 16), jnp.int32),
            pltpu.VMEM((4, 16), jnp.int32),
            pltpu.SemaphoreType.DMA,
            pltpu.SemaphoreType.DMA,
            pltpu.SemaphoreType.DMA,
            pltpu.SemaphoreType.DMA,
        ],
    )(xf, s1di, s2di)

    NF = 16
    FBLK = FFN // NF
    eos = pl.pallas_call(
        functools.partial(_ffn_body, NF),
        grid=(E, NF),
        in_specs=[
            pl.BlockSpec((CAP, H), lambda e, f: (e, 0)),
            pl.BlockSpec((1, H, FBLK), lambda e, f: (e, 0, f)),
            pl.BlockSpec((1, H, FBLK), lambda e, f: (e, 0, f)),
            pl.BlockSpec((1, FBLK, H), lambda e, f: (e, f, 0)),
        ],
        out_specs=pl.BlockSpec((CAP, H), lambda e, f: (e, 0)),
        out_shape=jax.ShapeDtypeStruct((NSLOT, H), jnp.float32),
        scratch_shapes=[
            pltpu.VMEM((CAP, H), jnp.float32),
        ],
    )(disp, w1, w3, w2)

    out = pl.kernel(
        _combine_body,
        out_type=jax.ShapeDtypeStruct((S, H), jnp.float32),
        mesh=mesh,
        scratch_types=[
            pltpu.VMEM((16, H), jnp.float32),
            pltpu.VMEM((16, H), jnp.float32),
            pltpu.VMEM((16,), jnp.int32),
            pltpu.VMEM((16,), jnp.int32),
            pltpu.VMEM((16, 16), jnp.float32),
            pltpu.VMEM((16, 16), jnp.float32),
            pltpu.SemaphoreType.DMA,
        ],
    )(eos, s1ci, s2ci, g1, g2)

    l_aux = small[1, 0]
    exp_counts = small[0, :E].astype(jnp.int32)
    return out.reshape(B, SS, H), l_aux, exp_counts


# speedup vs baseline: 1.0208x; 1.0097x over previous
import functools

import jax
import jax.numpy as jnp
from jax import lax
from jax.experimental import pallas as pl
from jax.experimental.pallas import tpu as pltpu
from jax.experimental.pallas import tpu_sc as plsc

HIDDEN = 2048
FFN = 8192
E = 8
S = 2048
CAP = 512
NSLOT = E * CAP
DUMP = NSLOT
DISP_ROWS = NSLOT + 8
NEG = -1e30

NW = 32
TW = S // NW


def _routing_body(xf_ref, wg_ref, small_ref, g1_ref, g2_ref,
                  s1c_ref, s2c_ref, s1d_ref, s2d_ref):
    xf = xf_ref[...]
    logits = jnp.dot(xf, wg_ref[...], preferred_element_type=jnp.float32)

    m = jnp.max(logits, axis=1, keepdims=True)
    ex = jnp.exp(logits - m)
    gates = ex / jnp.sum(ex, axis=1, keepdims=True)

    uppertri = (jax.lax.broadcasted_iota(jnp.int32, (E, E), 0)
                < jax.lax.broadcasted_iota(jnp.int32, (E, E), 1)).astype(jnp.float32)
    eq1 = (logits == m).astype(jnp.float32)
    before1 = jnp.dot(eq1, uppertri, preferred_element_type=jnp.float32)
    mask1 = eq1 * (before1 == 0.0).astype(jnp.float32)

    logits2 = jnp.where(mask1 > 0, NEG, logits)
    m2 = jnp.max(logits2, axis=1, keepdims=True)
    eq2 = (logits2 == m2).astype(jnp.float32)
    before2 = jnp.dot(eq2, uppertri, preferred_element_type=jnp.float32)
    mask2 = eq2 * (before2 == 0.0).astype(jnp.float32)

    lower = (jax.lax.broadcasted_iota(jnp.int32, (S, S), 0)
             >= jax.lax.broadcasted_iota(jnp.int32, (S, S), 1)).astype(jnp.float32)
    locations1 = jnp.dot(lower, mask1, preferred_element_type=jnp.float32) - 1.0
    count1 = jnp.sum(mask1, axis=0, keepdims=True)
    locations2 = (jnp.dot(lower, mask2, preferred_element_type=jnp.float32)
                  - 1.0 + count1)

    me = jnp.mean(gates, axis=0)
    ce = jnp.mean(mask1, axis=0)
    l_aux = jnp.mean(me * ce) * (E * E)

    mask1c = mask1 * (locations1 < CAP).astype(jnp.float32)
    mask2c = mask2 * (locations2 < CAP).astype(jnp.float32)
    loc1 = jnp.sum(locations1 * mask1c, axis=1, keepdims=True)
    loc2 = jnp.sum(locations2 * mask2c, axis=1, keepdims=True)
    g1 = jnp.sum(gates * mask1c, axis=1, keepdims=True)
    g2 = jnp.sum(gates * mask2c, axis=1, keepdims=True)
    denom = jnp.maximum(g1 + g2, jnp.finfo(jnp.float32).eps)
    g1_ref[...] = jnp.broadcast_to(g1 / denom, (S, 16))
    g2_ref[...] = jnp.broadcast_to(g2 / denom, (S, 16))

    lane = jax.lax.broadcasted_iota(jnp.int32, (S, E), 1).astype(jnp.float32)
    idx1 = jnp.sum(lane * mask1, axis=1, keepdims=True)
    idx2 = jnp.sum(lane * mask2, axis=1, keepdims=True)
    keep1 = jnp.sum(mask1c, axis=1, keepdims=True)
    keep2 = jnp.sum(mask2c, axis=1, keepdims=True)
    s1c = idx1 * CAP + loc1
    s2c = idx2 * CAP + loc2
    s1c_ref[...] = s1c
    s2c_ref[...] = s2c
    s1d_ref[...] = jnp.where(keep1 > 0, s1c, float(DUMP))
    s2d_ref[...] = jnp.where(keep2 > 0, s2c, float(DUMP))

    cnt_pad = jnp.concatenate(
        [count1, jnp.zeros((1, 128 - E), jnp.float32)], axis=1)
    row = jax.lax.broadcasted_iota(jnp.int32, (8, 128), 0)
    col = jax.lax.broadcasted_iota(jnp.int32, (8, 128), 1)
    small = jnp.where(row == 0, jnp.broadcast_to(cnt_pad, (8, 128)), 0.0)
    small = jnp.where((row == 1) & (col == 0), l_aux, small)
    small_ref[...] = small


def _dispatch_body(xf_hbm, s1d_hbm, s2d_hbm, disp_hbm,
                   xrows_v, i1_v, i2_v, gsem0, gsem1, ssem0, ssem1):
    wid = lax.axis_index("s") * 2 + lax.axis_index("c")
    nsub = 4
    sub_n = TW // nsub
    base = wid * TW
    gsem = (gsem0, gsem1)
    ssem = (ssem0, ssem1)
    pltpu.sync_copy(s1d_hbm.at[pl.ds(wid * nsub, nsub)], i1_v)
    pltpu.sync_copy(s2d_hbm.at[pl.ds(wid * nsub, nsub)], i2_v)

    def load(sub, buf):
        return pltpu.async_copy(
            xf_hbm.at[pl.ds(base + sub * sub_n, sub_n)],
            xrows_v.at[buf], gsem[buf])

    load_h = [load(0, 0), None]
    store_h = [None, None]
    for sub in range(nsub):
        buf = sub & 1
        if sub + 1 < nsub:
            ob = (sub + 1) & 1
            if store_h[ob] is not None:
                for st in store_h[ob]:
                    st.wait()
                store_h[ob] = None
            load_h[ob] = load(sub + 1, ob)
        load_h[buf].wait()
        store_h[buf] = (
            pltpu.async_copy(xrows_v.at[buf],
                             disp_hbm.at[i1_v.at[sub]], ssem[buf]),
            pltpu.async_copy(xrows_v.at[buf],
                             disp_hbm.at[i2_v.at[sub]], ssem[buf]),
        )
    for pair in store_h:
        if pair is not None:
            for st in pair:
                st.wait()


def _ffn_body(nf, disp_ref, w1_ref, w3_ref, w2_ref, eos_ref, acc):
    f = pl.program_id(1)

    d = disp_ref[...].astype(jnp.bfloat16)
    a = jnp.dot(d, w1_ref[0].astype(jnp.bfloat16),
                preferred_element_type=jnp.float32)
    b = jnp.dot(d, w3_ref[0].astype(jnp.bfloat16),
                preferred_element_type=jnp.float32)
    h = (a * jax.nn.sigmoid(a) * b).astype(jnp.bfloat16)
    contrib = jnp.dot(h, w2_ref[0].astype(jnp.bfloat16),
                      preferred_element_type=jnp.float32)

    @pl.when(f == 0)
    def _():
        acc[...] = jnp.zeros_like(acc)

    acc[...] += contrib

    @pl.when(f == nf - 1)
    def _():
        eos_ref[...] = acc[...]


def _combine_body(eos_hbm, s1c_hbm, s2c_hbm, g1_hbm, g2_hbm, out_hbm,
                  r1_v, r2_v, i1_v, i2_v, g1_v, g2_v, sem):
    wid = lax.axis_index("s") * 2 + lax.axis_index("c")
    nchunk = TW // 16
    base = wid * TW
    pltpu.sync_copy(s1c_hbm.at[pl.ds(wid * nchunk, nchunk)], i1_v)
    pltpu.sync_copy(s2c_hbm.at[pl.ds(wid * nchunk, nchunk)], i2_v)
    pltpu.sync_copy(g1_hbm.at[pl.ds(base, TW)], g1_v)
    pltpu.sync_copy(g2_hbm.at[pl.ds(base, TW)], g2_v)
    for chunk in range(nchunk):
        tb = base + chunk * 16
        gt1 = pltpu.async_copy(eos_hbm.at[i1_v.at[chunk]], r1_v, sem)
        gt2 = pltpu.async_copy(eos_hbm.at[i2_v.at[chunk]], r2_v, sem)
        gt1.wait()
        gt2.wait()

        def body_i(i, carry):
            bg1 = g1_v[chunk * 16 + i, :]
            bg2 = g2_v[chunk * 16 + i, :]
            for u in range(HIDDEN // 16):
                sl = pl.ds(u * 16, 16)
                r1_v[i, sl] = bg1 * r1_v[i, sl] + bg2 * r2_v[i, sl]
            return carry

        lax.fori_loop(0, 16, body_i, 0)
        pltpu.sync_copy(r1_v, out_hbm.at[pl.ds(tb, 16)])


def kernel(hidden_states, wg, w1, w3, w2):
    B, SS, H = hidden_states.shape
    xf = hidden_states.reshape(S, H)

    small, g1, g2, s1c, s2c, s1d, s2d = pl.pallas_call(
        _routing_body,
        out_shape=(
            jax.ShapeDtypeStruct((8, 128), jnp.float32),
            jax.ShapeDtypeStruct((S, 16), jnp.float32),
            jax.ShapeDtypeStruct((S, 16), jnp.float32),
            jax.ShapeDtypeStruct((S, 1), jnp.float32),
            jax.ShapeDtypeStruct((S, 1), jnp.float32),
            jax.ShapeDtypeStruct((S, 1), jnp.float32),
            jax.ShapeDtypeStruct((S, 1), jnp.float32),
        ),
    )(xf, wg)

    s1ci = s1c.reshape(S // 16, 16).astype(jnp.int32)
    s2ci = s2c.reshape(S // 16, 16).astype(jnp.int32)
    s1di = s1d.reshape(S // 16, 16).astype(jnp.int32)
    s2di = s2d.reshape(S // 16, 16).astype(jnp.int32)

    mesh = plsc.VectorSubcoreMesh(core_axis_name="c", subcore_axis_name="s")
    disp = pl.kernel(
        _dispatch_body,
        out_type=jax.ShapeDtypeStruct((DISP_ROWS, H), jnp.float32),
        mesh=mesh,
        scratch_types=[
            pltpu.VMEM((2, TW // 4, H), jnp.float32),
            pltpu.VMEM((4, 16), jnp.int32),
            pltpu.VMEM((4, 16), jnp.int32),
            pltpu.SemaphoreType.DMA,
            pltpu.SemaphoreType.DMA,
            pltpu.SemaphoreType.DMA,
            pltpu.SemaphoreType.DMA,
        ],
    )(xf, s1di, s2di)

    NF = 16
    FBLK = FFN // NF
    eos = pl.pallas_call(
        functools.partial(_ffn_body, NF),
        grid=(E, NF),
        in_specs=[
            pl.BlockSpec((CAP, H), lambda e, f: (e, 0)),
            pl.BlockSpec((1, H, FBLK), lambda e, f: (e, 0, f)),
            pl.BlockSpec((1, H, FBLK), lambda e, f: (e, 0, f)),
            pl.BlockSpec((1, FBLK, H), lambda e, f: (e, f, 0)),
        ],
        out_specs=pl.BlockSpec((CAP, H), lambda e, f: (e, 0)),
        out_shape=jax.ShapeDtypeStruct((NSLOT, H), jnp.float32),
        scratch_shapes=[
            pltpu.VMEM((CAP, H), jnp.float32),
        ],
    )(disp, w1, w3, w2)

    out = pl.kernel(
        _combine_body,
        out_type=jax.ShapeDtypeStruct((S, H), jnp.float32),
        mesh=mesh,
        scratch_types=[
            pltpu.VMEM((16, H), jnp.float32),
            pltpu.VMEM((16, H), jnp.float32),
            pltpu.VMEM((4, 16), jnp.int32),
            pltpu.VMEM((4, 16), jnp.int32),
            pltpu.VMEM((TW, 16), jnp.float32),
            pltpu.VMEM((TW, 16), jnp.float32),
            pltpu.SemaphoreType.DMA,
        ],
    )(eos, s1ci, s2ci, g1, g2)

    l_aux = small[1, 0]
    exp_counts = small[0, :E].astype(jnp.int32)
    return out.reshape(B, SS, H), l_aux, exp_counts
